# baseline (speedup 1.0000x reference)
import functools

import jax
import jax.numpy as jnp
from jax import lax
from jax.experimental import pallas as pl
from jax.experimental.pallas import tpu as pltpu
from jax.experimental.pallas import tpu_sc as plsc

VOCAB = 100000
HIDDEN = 1024
SEQ = 2048

_NC = 2
_NS = 16
_NW = _NC * _NS
_ROWS_PER_TILE = SEQ // _NW


def _sc_gather(table, ids):
    mesh = plsc.VectorSubcoreMesh(core_axis_name="c", subcore_axis_name="s")

    @functools.partial(
        pl.kernel,
        mesh=mesh,
        compiler_params=pltpu.CompilerParams(use_tc_tiling_on_sc=True),
        out_type=jax.ShapeDtypeStruct((SEQ, HIDDEN), jnp.float32),
        scratch_types=[
            pltpu.VMEM((_ROWS_PER_TILE,), jnp.int32),
            pltpu.VMEM((_ROWS_PER_TILE, HIDDEN), jnp.float32),
            pltpu.SemaphoreType.DMA,
        ],
    )
    def k(table_hbm, idx_hbm, out_hbm, idx_v, rows_v, sem):
        wid = lax.axis_index("s") * _NC + lax.axis_index("c")
        base = wid * _ROWS_PER_TILE
        pltpu.sync_copy(idx_hbm.at[pl.ds(base, _ROWS_PER_TILE)], idx_v)
        pltpu.async_copy(table_hbm.at[idx_v], rows_v, sem).wait()
        pltpu.sync_copy(rows_v, out_hbm.at[pl.ds(base, _ROWS_PER_TILE)])

    return k(table, ids)


_BV = 512
_NBLK = (VOCAB + _BV - 1) // _BV


def _proj_body(h_ref, w_ref, b_ref, out_ref):
    acc = lax.dot_general(
        h_ref[...], w_ref[...].astype(jnp.bfloat16),
        (((1,), (1,)), ((), ())),
        preferred_element_type=jnp.float32,
    )
    out_ref[...] = acc + b_ref[...]


def _tc_project(hidden, W_out, b2d):
    return pl.pallas_call(
        _proj_body,
        grid=(_NBLK,),
        in_specs=[
            pl.BlockSpec((SEQ, HIDDEN), lambda i: (0, 0)),
            pl.BlockSpec((_BV, HIDDEN), lambda i: (i, 0)),
            pl.BlockSpec((1, _BV), lambda i: (0, i)),
        ],
        out_specs=pl.BlockSpec((SEQ, _BV), lambda i: (0, i)),
        out_shape=jax.ShapeDtypeStruct((SEQ, VOCAB), jnp.float32),
    )(hidden, W_out, b2d)


def kernel(input_ids, embedding_table, W_out, b_out):
    ids = input_ids.reshape(SEQ).astype(jnp.int32)
    hidden = jnp.take(embedding_table, ids, axis=0)
    logits = _tc_project(hidden.astype(jnp.bfloat16), W_out, b_out.reshape(1, VOCAB))
    return (logits.reshape(1, SEQ, VOCAB), hidden.reshape(1, SEQ, HIDDEN))


# speedup vs baseline: 1.8628x; 1.8628x over previous
import functools

import jax
import jax.numpy as jnp
from jax import lax
from jax.experimental import pallas as pl
from jax.experimental.pallas import tpu as pltpu
from jax.experimental.pallas import tpu_sc as plsc

VOCAB = 100000
HIDDEN = 1024
SEQ = 2048

_NC = 2
_NS = 16
_NW = _NC * _NS
_ROWS_PER_TILE = SEQ // _NW


def _sc_gather(table, ids):
    mesh = plsc.VectorSubcoreMesh(core_axis_name="c", subcore_axis_name="s")

    @functools.partial(
        pl.kernel,
        mesh=mesh,
        out_type=jax.ShapeDtypeStruct((SEQ, HIDDEN), jnp.float32),
        scratch_types=[
            pltpu.VMEM((_ROWS_PER_TILE,), jnp.int32),
            pltpu.VMEM((_ROWS_PER_TILE, HIDDEN), jnp.float32),
            pltpu.SemaphoreType.DMA,
        ],
    )
    def k(table_hbm, idx_hbm, out_hbm, idx_v, rows_v, sem):
        wid = lax.axis_index("s") * _NC + lax.axis_index("c")
        base = wid * _ROWS_PER_TILE
        pltpu.sync_copy(idx_hbm.at[pl.ds(base, _ROWS_PER_TILE)], idx_v)
        pltpu.async_copy(table_hbm.at[idx_v], rows_v, sem).wait()
        pltpu.sync_copy(rows_v, out_hbm.at[pl.ds(base, _ROWS_PER_TILE)])

    return k(table, ids)


_BV = 512
_NBLK = (VOCAB + _BV - 1) // _BV


def _proj_body(h_ref, w_ref, b_ref, out_ref):
    acc = lax.dot_general(
        w_ref[...].astype(jnp.bfloat16), h_ref[...],
        (((1,), (1,)), ((), ())),
        preferred_element_type=jnp.float32,
    )
    out_ref[...] = acc + b_ref[...]


def _tc_project(hidden, W_out, bcol):
    return pl.pallas_call(
        _proj_body,
        grid=(_NBLK,),
        in_specs=[
            pl.BlockSpec((SEQ, HIDDEN), lambda i: (0, 0)),
            pl.BlockSpec((_BV, HIDDEN), lambda i: (i, 0)),
            pl.BlockSpec((_BV, 1), lambda i: (i, 0)),
        ],
        out_specs=pl.BlockSpec((_BV, SEQ), lambda i: (i, 0)),
        out_shape=jax.ShapeDtypeStruct((VOCAB, SEQ), jnp.float32),
    )(hidden, W_out, bcol)


def kernel(input_ids, embedding_table, W_out, b_out):
    ids = input_ids.reshape(SEQ).astype(jnp.int32)
    hidden = _sc_gather(embedding_table, ids)
    logits_t = _tc_project(
        hidden.astype(jnp.bfloat16), W_out, b_out.reshape(VOCAB, 1)
    )
    logits = logits_t.T.reshape(1, SEQ, VOCAB)
    return (logits, hidden.reshape(1, SEQ, HIDDEN))


# speedup vs baseline: 2.1986x; 1.1803x over previous
import functools

import jax
import jax.numpy as jnp
from jax import lax
from jax.experimental import pallas as pl
from jax.experimental.pallas import tpu as pltpu
from jax.experimental.pallas import tpu_sc as plsc

VOCAB = 100000
HIDDEN = 1024
SEQ = 2048

_NC = 2
_NS = 16
_NW = _NC * _NS
_ROWS_PER_TILE = SEQ // _NW
_HALF = _ROWS_PER_TILE // 2


def _sc_gather(table, ids):
    mesh = plsc.VectorSubcoreMesh(core_axis_name="c", subcore_axis_name="s")

    @functools.partial(
        pl.kernel,
        mesh=mesh,
        out_type=jax.ShapeDtypeStruct((SEQ, HIDDEN), jnp.float32),
        scratch_types=[
            pltpu.VMEM((_HALF,), jnp.int32),
            pltpu.VMEM((_HALF,), jnp.int32),
            pltpu.VMEM((_HALF, HIDDEN), jnp.float32),
            pltpu.VMEM((_HALF, HIDDEN), jnp.float32),
            pltpu.SemaphoreType.DMA,
            pltpu.SemaphoreType.DMA,
            pltpu.SemaphoreType.DMA,
            pltpu.SemaphoreType.DMA,
        ],
    )
    def k(table_hbm, idx_hbm, out_hbm, idx_a, idx_b, rows_a, rows_b,
          sga, sgb, ssa, ssb):
        wid = lax.axis_index("s") * _NC + lax.axis_index("c")
        base = wid * _ROWS_PER_TILE
        pltpu.sync_copy(idx_hbm.at[pl.ds(base, _HALF)], idx_a)
        ga = pltpu.async_copy(table_hbm.at[idx_a], rows_a, sga)
        pltpu.sync_copy(idx_hbm.at[pl.ds(base + _HALF, _HALF)], idx_b)
        gb = pltpu.async_copy(table_hbm.at[idx_b], rows_b, sgb)
        ga.wait()
        sa = pltpu.async_copy(rows_a, out_hbm.at[pl.ds(base, _HALF)], ssa)
        gb.wait()
        sb = pltpu.async_copy(rows_b, out_hbm.at[pl.ds(base + _HALF, _HALF)], ssb)
        sa.wait()
        sb.wait()

    return k(table, ids)


_BV = 512
_NBLK = (VOCAB + _BV - 1) // _BV


def _proj_body(h_ref, w_ref, b_ref, out_ref):
    acc = lax.dot_general(
        w_ref[...].astype(jnp.bfloat16), h_ref[...],
        (((1,), (1,)), ((), ())),
        preferred_element_type=jnp.float32,
    )
    out_ref[...] = acc + b_ref[...]


def _tc_project(hidden, W_out, bcol):
    return pl.pallas_call(
        _proj_body,
        grid=(_NBLK,),
        compiler_params=pltpu.CompilerParams(
            vmem_limit_bytes=100 * 1024 * 1024,
            allow_input_fusion=[True, False, False],
        ),
        in_specs=[
            pl.BlockSpec((SEQ, HIDDEN), lambda i: (0, 0)),
            pl.BlockSpec((_BV, HIDDEN), lambda i: (i, 0)),
            pl.BlockSpec((_BV, 1), lambda i: (i, 0)),
        ],
        out_specs=pl.BlockSpec((_BV, SEQ), lambda i: (i, 0)),
        out_shape=jax.ShapeDtypeStruct((VOCAB, SEQ), jnp.float32),
    )(hidden, W_out, bcol)


def kernel(input_ids, embedding_table, W_out, b_out):
    ids = input_ids.reshape(SEQ).astype(jnp.int32)
    hidden = _sc_gather(embedding_table, ids)
    logits_t = _tc_project(
        hidden.astype(jnp.bfloat16), W_out, b_out.reshape(VOCAB, 1)
    )
    logits = logits_t.T.reshape(1, SEQ, VOCAB)
    return (logits, hidden.reshape(1, SEQ, HIDDEN))


# speedup vs baseline: 2.2010x; 1.0011x over previous
import functools

import jax
import jax.numpy as jnp
from jax import lax
from jax.experimental import pallas as pl
from jax.experimental.pallas import tpu as pltpu
from jax.experimental.pallas import tpu_sc as plsc

VOCAB = 100000
HIDDEN = 1024
SEQ = 2048

_NC = 2
_NS = 16
_NW = _NC * _NS
_ROWS_PER_TILE = SEQ // _NW


def _sc_gather(table, ids):
    mesh = plsc.VectorSubcoreMesh(core_axis_name="c", subcore_axis_name="s")

    @functools.partial(
        pl.kernel,
        mesh=mesh,
        out_type=jax.ShapeDtypeStruct((SEQ, HIDDEN), jnp.float32),
        scratch_types=[
            pltpu.VMEM((_ROWS_PER_TILE,), jnp.int32),
            pltpu.VMEM((_ROWS_PER_TILE, HIDDEN), jnp.float32),
            pltpu.SemaphoreType.DMA,
        ],
    )
    def k(table_hbm, idx_hbm, out_hbm, idx_v, rows_v, sem):
        wid = lax.axis_index("s") * _NC + lax.axis_index("c")
        base = wid * _ROWS_PER_TILE
        pltpu.sync_copy(idx_hbm.at[pl.ds(base, _ROWS_PER_TILE)], idx_v)
        pltpu.async_copy(table_hbm.at[idx_v], rows_v, sem).wait()
        pltpu.sync_copy(rows_v, out_hbm.at[pl.ds(base, _ROWS_PER_TILE)])

    return k(table, ids)


_BV = 512
_NBLK = (VOCAB + _BV - 1) // _BV


def _proj_body(h_ref, w_ref, b_ref, out_ref):
    acc = lax.dot_general(
        w_ref[...].astype(jnp.bfloat16), h_ref[...],
        (((1,), (1,)), ((), ())),
        preferred_element_type=jnp.float32,
    )
    out_ref[...] = acc + b_ref[...]


def _tc_project(hidden, W_out, bcol):
    return pl.pallas_call(
        _proj_body,
        grid=(_NBLK,),
        compiler_params=pltpu.CompilerParams(
            vmem_limit_bytes=100 * 1024 * 1024,
            allow_input_fusion=[True, False, False],
            dimension_semantics=("parallel",),
        ),
        in_specs=[
            pl.BlockSpec((SEQ, HIDDEN), lambda i: (0, 0)),
            pl.BlockSpec((_BV, HIDDEN), lambda i: (i, 0)),
            pl.BlockSpec((_BV, 1), lambda i: (i, 0)),
        ],
        out_specs=pl.BlockSpec((_BV, SEQ), lambda i: (i, 0)),
        out_shape=jax.ShapeDtypeStruct((VOCAB, SEQ), jnp.float32),
    )(hidden, W_out, bcol)


def kernel(input_ids, embedding_table, W_out, b_out):
    ids = input_ids.reshape(SEQ).astype(jnp.int32)
    hidden = _sc_gather(embedding_table, ids)
    logits_t = _tc_project(
        hidden.astype(jnp.bfloat16), W_out, b_out.reshape(VOCAB, 1)
    )
    logits = logits_t.T.reshape(1, SEQ, VOCAB)
    return (logits, hidden.reshape(1, SEQ, HIDDEN))


# speedup vs baseline: 2.2025x; 1.0007x over previous
import functools

import jax
import jax.numpy as jnp
from jax import lax
from jax.experimental import pallas as pl
from jax.experimental.pallas import tpu as pltpu
from jax.experimental.pallas import tpu_sc as plsc

VOCAB = 100000
HIDDEN = 1024
SEQ = 2048

_NC = 2
_NS = 16
_NW = _NC * _NS
_ROWS_PER_TILE = SEQ // _NW


def _sc_gather(table, ids):
    mesh = plsc.VectorSubcoreMesh(core_axis_name="c", subcore_axis_name="s")

    @functools.partial(
        pl.kernel,
        mesh=mesh,
        out_type=jax.ShapeDtypeStruct((SEQ, HIDDEN), jnp.float32),
        scratch_types=[
            pltpu.VMEM((_ROWS_PER_TILE,), jnp.int32),
            pltpu.VMEM((_ROWS_PER_TILE, HIDDEN), jnp.float32),
            pltpu.SemaphoreType.DMA,
        ],
    )
    def k(table_hbm, idx_hbm, out_hbm, idx_v, rows_v, sem):
        wid = lax.axis_index("s") * _NC + lax.axis_index("c")
        base = wid * _ROWS_PER_TILE
        pltpu.sync_copy(idx_hbm.at[pl.ds(base, _ROWS_PER_TILE)], idx_v)
        pltpu.async_copy(table_hbm.at[idx_v], rows_v, sem).wait()
        pltpu.sync_copy(rows_v, out_hbm.at[pl.ds(base, _ROWS_PER_TILE)])

    return k(table, ids)


_BV = 512
_NBLK = (VOCAB + _BV - 1) // _BV


def _proj_body(h_ref, w_ref, b_ref, out_ref):
    acc = lax.dot_general(
        w_ref[...].astype(jnp.bfloat16), h_ref[...],
        (((1,), (1,)), ((), ())),
        preferred_element_type=jnp.float32,
    )
    out_ref[...] = acc + b_ref[...]


def _tc_project(hidden, W_out, bcol):
    return pl.pallas_call(
        _proj_body,
        grid=(_NBLK,),
        compiler_params=pltpu.CompilerParams(
            vmem_limit_bytes=100 * 1024 * 1024,
            allow_input_fusion=[True, False, False],
        ),
        in_specs=[
            pl.BlockSpec((SEQ, HIDDEN), lambda i: (0, 0)),
            pl.BlockSpec((_BV, HIDDEN), lambda i: (i, 0)),
            pl.BlockSpec((_BV, 1), lambda i: (i, 0)),
        ],
        out_specs=pl.BlockSpec((_BV, SEQ), lambda i: (i, 0)),
        out_shape=jax.ShapeDtypeStruct((VOCAB, SEQ), jnp.float32),
    )(hidden, W_out, bcol)


def kernel(input_ids, embedding_table, W_out, b_out):
    ids = input_ids.reshape(SEQ).astype(jnp.int32)
    hidden = _sc_gather(embedding_table, ids)
    logits_t = _tc_project(
        hidden.astype(jnp.bfloat16), W_out, b_out.reshape(VOCAB, 1)
    )
    logits = logits_t.T.reshape(1, SEQ, VOCAB)
    return (logits, hidden.reshape(1, SEQ, HIDDEN))
